# baseline (speedup 1.0000x reference)
import functools

import jax
import jax.numpy as jnp
from jax import lax
from jax.experimental import pallas as pl
from jax.experimental.pallas import tpu as pltpu
from jax.experimental.pallas import tpu_sc as plsc

N = 10000
E = 320000
H = 128
G = 128
L = 3

NC = 2
NS = 16
NW = NC * NS
LANES = 16

CH = 128
EPC = 80
NCHUNKS_P = NW * EPC
E_PAD = NCHUNKS_P * CH
SLAB = 40
NSPARE = 8
NA = N + NSPARE
DEGW = 128

CHP = 200
NPCHUNKS = N // CHP
G_PER_SUB = G // NS

def _fill_rows(buf, nrows, ngroups, val):
    v = jnp.full((LANES,), val, dtype=buf.dtype)

    @pl.loop(0, nrows)
    def _(r):
        @pl.loop(0, ngroups)
        def _(c):
            buf[r, pl.ds(c * LANES, LANES)] = v


@functools.lru_cache(maxsize=None)
def _sc_kernels():
    mesh = plsc.VectorSubcoreMesh(core_axis_name="c", subcore_axis_name="s")

    @functools.partial(
        pl.kernel,
        out_type=jax.ShapeDtypeStruct((NC, N, DEGW), jnp.float32),
        mesh=mesh,
        scratch_types=[
            pltpu.VMEM((SLAB, CH), jnp.int32),
            pltpu.VMEM((CH, DEGW), jnp.float32),
            pltpu.SemaphoreType.DMA,
            pltpu.SemaphoreType.DMA,
            pltpu.VMEM_SHARED((NA, DEGW), jnp.float32),
        ],
    )
    def sc_degree(e3_hbm, out_hbm, dst_v, ones_v, sem0, sem1, deg_sh):
        _sc_degree_body(e3_hbm, out_hbm, dst_v, ones_v, (sem0, sem1), deg_sh)

    @functools.partial(
        pl.kernel,
        out_type=jax.ShapeDtypeStruct((NC, N, H), jnp.float32),
        mesh=mesh,
        scratch_types=[
            pltpu.VMEM((2, SLAB, CH), jnp.int32),
            pltpu.VMEM((CH, H), jnp.float32),
            pltpu.VMEM((CH, H), jnp.float32),
            pltpu.SemaphoreType.DMA,
            pltpu.SemaphoreType.DMA,
            pltpu.SemaphoreType.DMA,
            pltpu.SemaphoreType.DMA,
            pltpu.VMEM_SHARED((NA, H), jnp.float32),
        ],
    )
    def sc_edge_agg(m2_hbm, e3_hbm, out_hbm, idx_v, rows0, rows1, sem0, sem1,
                    ssem0, ssem1, agg_sh):
        _sc_edge_agg_body(m2_hbm, e3_hbm, out_hbm, idx_v, (rows0, rows1),
                          (sem0, sem1), (ssem0, ssem1), agg_sh)

    @functools.partial(
        pl.kernel,
        out_type=[
            jax.ShapeDtypeStruct((NC, G, H), jnp.float32),
            jax.ShapeDtypeStruct((NC, G, H), jnp.float32),
        ],
        mesh=mesh,
        scratch_types=[
            pltpu.VMEM((CHP,), jnp.int32),
            pltpu.VMEM((CHP, H), jnp.float32),
            pltpu.VMEM((CHP, H), jnp.float32),
            pltpu.VMEM_SHARED((G, H), jnp.float32),
            pltpu.VMEM_SHARED((G, H), jnp.float32),
        ],
    )
    def sc_pool(h_hbm, batch_hbm, sum_hbm, cnt_hbm, batch_v, rows_v, ones_v,
                sum_sh, cnt_sh):
        _sc_pool_body(h_hbm, batch_hbm, sum_hbm, cnt_hbm, batch_v, rows_v,
                      ones_v, sum_sh, cnt_sh)

    return sc_degree, sc_edge_agg, sc_pool


def _zero_accum(src_buf, acc_sh, sid, nrows):
    nfull = nrows // CH

    @pl.loop(0, -(-nfull // NS))
    def _(t):
        zc = sid + NS * t

        @pl.when(zc < nfull)
        def _():
            pltpu.sync_copy(src_buf, acc_sh.at[pl.ds(zc * CH, CH)])

    tail = nrows - nfull * CH
    if tail:
        @pl.when(sid == 0)
        def _():
            pltpu.sync_copy(src_buf.at[pl.ds(0, tail)],
                            acc_sh.at[pl.ds(nfull * CH, tail)])


def _writeout_accum(acc_sh, out_hbm, cid, sid, nrows):
    nfull = nrows // CH

    @pl.loop(0, -(-nfull // NS))
    def _(t):
        zc = sid + NS * t

        @pl.when(zc < nfull)
        def _():
            pltpu.sync_copy(acc_sh.at[pl.ds(zc * CH, CH)],
                            out_hbm.at[cid, pl.ds(zc * CH, CH)])

    tail = nrows - nfull * CH
    if tail:
        @pl.when(sid == 0)
        def _():
            pltpu.sync_copy(acc_sh.at[pl.ds(nfull * CH, tail)],
                            out_hbm.at[cid, pl.ds(nfull * CH, tail)])


def _sc_degree_body(e3_hbm, out_hbm, dst_v, ones_v, sems, deg_sh):
    cid = lax.axis_index("c")
    sid = lax.axis_index("s")
    wid = sid * NC + cid

    _fill_rows(ones_v, CH, DEGW // LANES, 0.0)
    _zero_accum(ones_v, deg_sh, sid, NA)
    plsc.subcore_barrier()
    _fill_rows(ones_v, CH, DEGW // LANES, 1.0)

    def scat_start(b, k):
        pltpu.async_copy(ones_v, deg_sh.at[dst_v.at[k]], sems[b], add=True)

    def scat_wait(b, k):
        pltpu.make_async_copy(ones_v, deg_sh.at[dst_v.at[k]], sems[b]).wait()

    base = wid * EPC
    for s in range(EPC // SLAB):
        pltpu.sync_copy(e3_hbm.at[1, pl.ds(base + s * SLAB, SLAB)], dst_v)
        scat_start(0, 0)
        scat_start(1, 1)

        @pl.loop(0, SLAB // 2 - 1)
        def _(j):
            k = 2 * j
            scat_wait(0, k)
            scat_start(0, k + 2)
            scat_wait(1, k + 1)
            scat_start(1, k + 3)

        scat_wait(0, SLAB - 2)
        scat_wait(1, SLAB - 1)

    plsc.subcore_barrier()
    _writeout_accum(deg_sh, out_hbm, cid, sid, N)


def _sc_edge_agg_body(m2_hbm, e3_hbm, out_hbm, idx_v, rows, sems, ssems,
                      agg_sh):
    cid = lax.axis_index("c")
    sid = lax.axis_index("s")
    wid = sid * NC + cid

    _fill_rows(rows[0], CH, H // LANES, 0.0)
    _zero_accum(rows[0], agg_sh, sid, NA)
    plsc.subcore_barrier()

    def gat(b, k):
        return pltpu.make_async_copy(m2_hbm.at[idx_v.at[0, k]], rows[b],
                                     sems[b])

    def sct_start(b, k):
        pltpu.async_copy(rows[b], agg_sh.at[idx_v.at[1, k]], ssems[b],
                         add=True)

    def sct_wait(b, k):
        pltpu.make_async_copy(rows[b], agg_sh.at[idx_v.at[1, k]],
                              ssems[b]).wait()

    base = wid * EPC
    for s in range(EPC // SLAB):
        pltpu.sync_copy(e3_hbm.at[:, pl.ds(base + s * SLAB, SLAB)], idx_v)
        gat(0, 0).start()
        gat(1, 1).start()
        gat(0, 0).wait()
        sct_start(0, 0)

        @pl.loop(0, SLAB // 2 - 1)
        def _(j):
            k = 2 * j
            sct_wait(0, k)
            gat(0, k + 2).start()
            gat(1, k + 1).wait()
            sct_start(1, k + 1)
            gat(0, k + 2).wait()
            sct_start(0, k + 2)
            sct_wait(1, k + 1)
            gat(1, k + 3).start()

        sct_wait(0, SLAB - 2)
        gat(1, SLAB - 1).wait()
        sct_start(1, SLAB - 1)
        sct_wait(1, SLAB - 1)

    plsc.subcore_barrier()
    _writeout_accum(agg_sh, out_hbm, cid, sid, N)


def _sc_pool_body(h_hbm, batch_hbm, sum_hbm, cnt_hbm, batch_v, rows_v, ones_v,
                  sum_sh, cnt_sh):
    cid = lax.axis_index("c")
    sid = lax.axis_index("s")
    wid = sid * NC + cid

    _fill_rows(rows_v, G_PER_SUB, H // LANES, 0.0)
    _fill_rows(ones_v, G_PER_SUB, H // LANES, 0.0)
    pltpu.sync_copy(rows_v.at[pl.ds(0, G_PER_SUB)],
                    sum_sh.at[pl.ds(sid * G_PER_SUB, G_PER_SUB)])
    pltpu.sync_copy(ones_v.at[pl.ds(0, G_PER_SUB)],
                    cnt_sh.at[pl.ds(sid * G_PER_SUB, G_PER_SUB)])
    plsc.subcore_barrier()
    _fill_rows(ones_v, CHP, H // LANES, 1.0)

    nper = NPCHUNKS // NW

    @pl.loop(0, nper + 1)
    def _(j):
        chunk = wid + NW * j

        @pl.when(chunk < NPCHUNKS)
        def _():
            base = chunk * CHP
            pltpu.sync_copy(h_hbm.at[pl.ds(base, CHP)], rows_v)
            pltpu.sync_copy(batch_hbm.at[pl.ds(base, CHP)], batch_v)
            pltpu.sync_copy(rows_v, sum_sh.at[batch_v], add=True)
            pltpu.sync_copy(ones_v, cnt_sh.at[batch_v], add=True)

    plsc.subcore_barrier()
    pltpu.sync_copy(sum_sh.at[pl.ds(sid * G_PER_SUB, G_PER_SUB)],
                    sum_hbm.at[cid, pl.ds(sid * G_PER_SUB, G_PER_SUB)])
    pltpu.sync_copy(cnt_sh.at[pl.ds(sid * G_PER_SUB, G_PER_SUB)],
                    cnt_hbm.at[cid, pl.ds(sid * G_PER_SUB, G_PER_SUB)])


_BN = 1000


def _dinv_body(d_ref, o_ref):
    deg = d_ref[0, :, 0:1] + d_ref[1, :, 0:1] + 1.0
    o_ref[...] = lax.rsqrt(deg)


def _tc_dinv(deg_parts):
    return pl.pallas_call(
        _dinv_body,
        grid=(N // _BN,),
        in_specs=[pl.BlockSpec((NC, _BN, DEGW), lambda i: (0, i, 0))],
        out_specs=pl.BlockSpec((_BN, 1), lambda i: (i, 0)),
        out_shape=jax.ShapeDtypeStruct((N, 1), jnp.float32),
    )(deg_parts)


def _projmm_body(x_ref, wi_ref, bi_ref, w_ref, s_ref, ho_ref, mo_ref):
    h = jnp.dot(x_ref[...], wi_ref[...],
                preferred_element_type=jnp.float32) + bi_ref[...]
    ho_ref[...] = h
    mo_ref[...] = jnp.dot(h, w_ref[...],
                          preferred_element_type=jnp.float32) * s_ref[...]


def _tc_projmm(x, w_in, b_in2, w0, dinv):
    return pl.pallas_call(
        _projmm_body,
        grid=(N // _BN,),
        in_specs=[
            pl.BlockSpec((_BN, H), lambda i: (i, 0)),
            pl.BlockSpec((H, H), lambda i: (0, 0)),
            pl.BlockSpec((1, H), lambda i: (0, 0)),
            pl.BlockSpec((H, H), lambda i: (0, 0)),
            pl.BlockSpec((_BN, 1), lambda i: (i, 0)),
        ],
        out_specs=[pl.BlockSpec((_BN, H), lambda i: (i, 0)),
                   pl.BlockSpec((_BN, H), lambda i: (i, 0))],
        out_shape=[jax.ShapeDtypeStruct((N, H), jnp.float32),
                   jax.ShapeDtypeStruct((N, H), jnp.float32)],
    )(x, w_in, b_in2, w0, dinv)


def _ln_relu_res(a_ref, m_ref, s_ref, cb_ref, g_ref, b_ref, hp_ref):
    t = (a_ref[0] + a_ref[1] + m_ref[...]) * s_ref[...] + cb_ref[...]
    mu = jnp.mean(t, axis=-1, keepdims=True)
    var = jnp.mean((t - mu) ** 2, axis=-1, keepdims=True)
    hn = (t - mu) * lax.rsqrt(var + 1e-5) * g_ref[...] + b_ref[...]
    return jnp.maximum(hn, 0.0) + hp_ref[...]


def _postmm_body(a_ref, m_ref, s_ref, cb_ref, g_ref, b_ref, hp_ref, w_ref,
                 ho_ref, mo_ref):
    h_new = _ln_relu_res(a_ref, m_ref, s_ref, cb_ref, g_ref, b_ref, hp_ref)
    ho_ref[...] = h_new
    mo_ref[...] = jnp.dot(h_new, w_ref[...],
                          preferred_element_type=jnp.float32) * s_ref[...]


def _tc_postmm(agg_parts, m2, dinv, conv_b_i, ln_g_i, ln_b_i, h_prev, w_next):
    return pl.pallas_call(
        _postmm_body,
        grid=(N // _BN,),
        in_specs=[
            pl.BlockSpec((NC, _BN, H), lambda i: (0, i, 0)),
            pl.BlockSpec((_BN, H), lambda i: (i, 0)),
            pl.BlockSpec((_BN, 1), lambda i: (i, 0)),
            pl.BlockSpec((1, H), lambda i: (0, 0)),
            pl.BlockSpec((1, H), lambda i: (0, 0)),
            pl.BlockSpec((1, H), lambda i: (0, 0)),
            pl.BlockSpec((_BN, H), lambda i: (i, 0)),
            pl.BlockSpec((H, H), lambda i: (0, 0)),
        ],
        out_specs=[pl.BlockSpec((_BN, H), lambda i: (i, 0)),
                   pl.BlockSpec((_BN, H), lambda i: (i, 0))],
        out_shape=[jax.ShapeDtypeStruct((N, H), jnp.float32),
                   jax.ShapeDtypeStruct((N, H), jnp.float32)],
    )(agg_parts, m2, dinv, conv_b_i, ln_g_i, ln_b_i, h_prev, w_next)


def _post_body(a_ref, m_ref, s_ref, cb_ref, g_ref, b_ref, hp_ref, o_ref):
    o_ref[...] = _ln_relu_res(a_ref, m_ref, s_ref, cb_ref, g_ref, b_ref,
                              hp_ref)


def _tc_post(agg_parts, m2, dinv, conv_b_i, ln_g_i, ln_b_i, h_prev):
    return pl.pallas_call(
        _post_body,
        grid=(N // _BN,),
        in_specs=[
            pl.BlockSpec((NC, _BN, H), lambda i: (0, i, 0)),
            pl.BlockSpec((_BN, H), lambda i: (i, 0)),
            pl.BlockSpec((_BN, 1), lambda i: (i, 0)),
            pl.BlockSpec((1, H), lambda i: (0, 0)),
            pl.BlockSpec((1, H), lambda i: (0, 0)),
            pl.BlockSpec((1, H), lambda i: (0, 0)),
            pl.BlockSpec((_BN, H), lambda i: (i, 0)),
        ],
        out_specs=pl.BlockSpec((_BN, H), lambda i: (i, 0)),
        out_shape=jax.ShapeDtypeStruct((N, H), jnp.float32),
    )(agg_parts, m2, dinv, conv_b_i, ln_g_i, ln_b_i, h_prev)


def _mlp_body(pp_ref, cp_ref, w1_ref, b1_ref, w2_ref, b2_ref, o_ref):
    sums = pp_ref[0] + pp_ref[1]
    cnts = cp_ref[0, :, 0:1] + cp_ref[1, :, 0:1]
    pooled = sums / jnp.maximum(cnts, 1.0)
    z = jnp.dot(pooled, w1_ref[...], preferred_element_type=jnp.float32)
    z = jnp.maximum(z + b1_ref[...], 0.0)
    o_ref[...] = jnp.dot(z, w2_ref[...],
                         preferred_element_type=jnp.float32) + b2_ref[...]


def _tc_mlp(pool_parts, cnt_parts, w1, b1, w2, b2):
    h2 = w1.shape[1]
    return pl.pallas_call(
        _mlp_body,
        grid=(1,),
        in_specs=[
            pl.BlockSpec((NC, G, H), lambda i: (0, 0, 0)),
            pl.BlockSpec((NC, G, H), lambda i: (0, 0, 0)),
            pl.BlockSpec((H, h2), lambda i: (0, 0)),
            pl.BlockSpec((1, h2), lambda i: (0, 0)),
            pl.BlockSpec((h2, 1), lambda i: (0, 0)),
            pl.BlockSpec((1, 1), lambda i: (0, 0)),
        ],
        out_specs=pl.BlockSpec((G, 1), lambda i: (0, 0)),
        out_shape=jax.ShapeDtypeStruct((G, 1), jnp.float32),
    )(pool_parts, cnt_parts, w1, b1, w2, b2)


def kernel(x, edge_index, batch, W_in, b_in, conv_W, conv_b, ln_g, ln_b,
           W1, b1, W2, b2):
    _sc_degree, _sc_edge_agg, _sc_pool = _sc_kernels()

    k = jnp.arange(E_PAD - E, dtype=jnp.int32)
    pad = jnp.stack([k % 1024, N + (k % NSPARE)])
    e3 = (jnp.concatenate([edge_index, pad], axis=1)
          .reshape(2, EPC, NW, CH).swapaxes(1, 2)
          .reshape(2, NCHUNKS_P, CH))

    deg_parts = _sc_degree(e3)
    dinv = _tc_dinv(deg_parts)

    h, m2 = _tc_projmm(x, W_in, b_in.reshape(1, H), conv_W[0], dinv)
    for i in range(L):
        agg_parts = _sc_edge_agg(m2, e3)
        args = (agg_parts, m2, dinv, conv_b[i].reshape(1, H),
                ln_g[i].reshape(1, H), ln_b[i].reshape(1, H), h)
        if i < L - 1:
            h, m2 = _tc_postmm(*args, conv_W[i + 1])
        else:
            h = _tc_post(*args)

    pool_parts, cnt_parts = _sc_pool(h, batch)
    out = _tc_mlp(pool_parts, cnt_parts, W1, b1.reshape(1, -1), W2,
                  b2.reshape(1, 1))
    return out


# speedup vs baseline: 1.1948x; 1.1948x over previous
import functools

import jax
import jax.numpy as jnp
from jax import lax
from jax.experimental import pallas as pl
from jax.experimental.pallas import tpu as pltpu
from jax.experimental.pallas import tpu_sc as plsc

N = 10000
E = 320000
H = 128
G = 128
L = 3

NC = 2
NS = 16
NW = NC * NS
LANES = 16

CH = 128
EPC = 80
NCHUNKS_P = NW * EPC
E_PAD = NCHUNKS_P * CH
SLAB = 40
NSPARE = 8
NA = N + NSPARE
DEGW = 128

CHP = 200
NPCHUNKS = N // CHP
G_PER_SUB = G // NS

def _fill_rows(buf, nrows, ngroups, val):
    v = jnp.full((LANES,), val, dtype=buf.dtype)

    @pl.loop(0, nrows)
    def _(r):
        @pl.loop(0, ngroups)
        def _(c):
            buf[r, pl.ds(c * LANES, LANES)] = v


@functools.lru_cache(maxsize=None)
def _sc_kernels():
    mesh = plsc.VectorSubcoreMesh(core_axis_name="c", subcore_axis_name="s")

    @functools.partial(
        pl.kernel,
        out_type=jax.ShapeDtypeStruct((NC, N, DEGW), jnp.float32),
        mesh=mesh,
        scratch_types=[
            pltpu.VMEM((SLAB, CH), jnp.int32),
            pltpu.VMEM((CH, DEGW), jnp.float32),
            pltpu.SemaphoreType.DMA,
            pltpu.SemaphoreType.DMA,
            pltpu.VMEM_SHARED((NA, DEGW), jnp.float32),
        ],
    )
    def sc_degree(e3_hbm, out_hbm, dst_v, ones_v, sem0, sem1, deg_sh):
        _sc_degree_body(e3_hbm, out_hbm, dst_v, ones_v, (sem0, sem1), deg_sh)

    @functools.partial(
        pl.kernel,
        out_type=jax.ShapeDtypeStruct((NC, N, H), jnp.float32),
        mesh=mesh,
        scratch_types=[
            pltpu.VMEM((2, SLAB, CH), jnp.int32),
            pltpu.VMEM((CH, H), jnp.float32),
            pltpu.VMEM((CH, H), jnp.float32),
            pltpu.SemaphoreType.DMA,
            pltpu.SemaphoreType.DMA,
            pltpu.VMEM_SHARED((NA, H), jnp.float32),
        ],
    )
    def sc_edge_agg(m2_hbm, e3_hbm, out_hbm, idx_v, rows0, rows1, sem0, sem1,
                    agg_sh):
        _sc_edge_agg_body(m2_hbm, e3_hbm, out_hbm, idx_v, (rows0, rows1),
                          (sem0, sem1), agg_sh)

    @functools.partial(
        pl.kernel,
        out_type=[
            jax.ShapeDtypeStruct((NC, G, H), jnp.float32),
            jax.ShapeDtypeStruct((NC, G, H), jnp.float32),
        ],
        mesh=mesh,
        scratch_types=[
            pltpu.VMEM((CHP,), jnp.int32),
            pltpu.VMEM((CHP, H), jnp.float32),
            pltpu.VMEM((CHP, H), jnp.float32),
            pltpu.VMEM_SHARED((G, H), jnp.float32),
            pltpu.VMEM_SHARED((G, H), jnp.float32),
        ],
    )
    def sc_pool(h_hbm, batch_hbm, sum_hbm, cnt_hbm, batch_v, rows_v, ones_v,
                sum_sh, cnt_sh):
        _sc_pool_body(h_hbm, batch_hbm, sum_hbm, cnt_hbm, batch_v, rows_v,
                      ones_v, sum_sh, cnt_sh)

    return sc_degree, sc_edge_agg, sc_pool


def _zero_accum(src_buf, acc_sh, sid, nrows):
    nfull = nrows // CH

    @pl.loop(0, -(-nfull // NS))
    def _(t):
        zc = sid + NS * t

        @pl.when(zc < nfull)
        def _():
            pltpu.sync_copy(src_buf, acc_sh.at[pl.ds(zc * CH, CH)])

    tail = nrows - nfull * CH
    if tail:
        @pl.when(sid == 0)
        def _():
            pltpu.sync_copy(src_buf.at[pl.ds(0, tail)],
                            acc_sh.at[pl.ds(nfull * CH, tail)])


def _writeout_accum(acc_sh, out_hbm, cid, sid, nrows):
    nfull = nrows // CH

    @pl.loop(0, -(-nfull // NS))
    def _(t):
        zc = sid + NS * t

        @pl.when(zc < nfull)
        def _():
            pltpu.sync_copy(acc_sh.at[pl.ds(zc * CH, CH)],
                            out_hbm.at[cid, pl.ds(zc * CH, CH)])

    tail = nrows - nfull * CH
    if tail:
        @pl.when(sid == 0)
        def _():
            pltpu.sync_copy(acc_sh.at[pl.ds(nfull * CH, tail)],
                            out_hbm.at[cid, pl.ds(nfull * CH, tail)])


def _sc_degree_body(e3_hbm, out_hbm, dst_v, ones_v, sems, deg_sh):
    cid = lax.axis_index("c")
    sid = lax.axis_index("s")
    wid = sid * NC + cid

    _fill_rows(ones_v, CH, DEGW // LANES, 0.0)
    _zero_accum(ones_v, deg_sh, sid, NA)
    plsc.subcore_barrier()
    _fill_rows(ones_v, CH, DEGW // LANES, 1.0)

    def scat_start(b, k):
        pltpu.async_copy(ones_v, deg_sh.at[dst_v.at[k]], sems[b], add=True)

    def scat_wait(b, k):
        pltpu.make_async_copy(ones_v, deg_sh.at[dst_v.at[k]], sems[b]).wait()

    base = wid * EPC
    for s in range(EPC // SLAB):
        pltpu.sync_copy(e3_hbm.at[1, pl.ds(base + s * SLAB, SLAB)], dst_v)
        scat_start(0, 0)
        scat_start(1, 1)

        @pl.loop(0, SLAB // 2 - 1)
        def _(j):
            k = 2 * j
            scat_wait(0, k)
            scat_start(0, k + 2)
            scat_wait(1, k + 1)
            scat_start(1, k + 3)

        scat_wait(0, SLAB - 2)
        scat_wait(1, SLAB - 1)

    plsc.subcore_barrier()
    _writeout_accum(deg_sh, out_hbm, cid, sid, N)


def _sc_edge_agg_body(m2_hbm, e3_hbm, out_hbm, idx_v, rows, sems, agg_sh):
    cid = lax.axis_index("c")
    sid = lax.axis_index("s")
    wid = sid * NC + cid

    _fill_rows(rows[0], CH, H // LANES, 0.0)
    _zero_accum(rows[0], agg_sh, sid, NA)
    plsc.subcore_barrier()

    def gat(b, k):
        return pltpu.make_async_copy(m2_hbm.at[idx_v.at[0, k]], rows[b],
                                     sems[b])

    def fin(b, k):
        gat(b, k).wait()
        pltpu.sync_copy(rows[b], agg_sh.at[idx_v.at[1, k]], add=True)

    base = wid * EPC
    for s in range(EPC // SLAB):
        pltpu.sync_copy(e3_hbm.at[:, pl.ds(base + s * SLAB, SLAB)], idx_v)
        gat(0, 0).start()
        gat(1, 1).start()

        @pl.loop(0, SLAB // 2 - 1)
        def _(j):
            k = 2 * j
            fin(0, k)
            gat(0, k + 2).start()
            fin(1, k + 1)
            gat(1, k + 3).start()

        fin(0, SLAB - 2)
        fin(1, SLAB - 1)

    plsc.subcore_barrier()
    _writeout_accum(agg_sh, out_hbm, cid, sid, N)


def _sc_pool_body(h_hbm, batch_hbm, sum_hbm, cnt_hbm, batch_v, rows_v, ones_v,
                  sum_sh, cnt_sh):
    cid = lax.axis_index("c")
    sid = lax.axis_index("s")
    wid = sid * NC + cid

    _fill_rows(rows_v, G_PER_SUB, H // LANES, 0.0)
    _fill_rows(ones_v, G_PER_SUB, H // LANES, 0.0)
    pltpu.sync_copy(rows_v.at[pl.ds(0, G_PER_SUB)],
                    sum_sh.at[pl.ds(sid * G_PER_SUB, G_PER_SUB)])
    pltpu.sync_copy(ones_v.at[pl.ds(0, G_PER_SUB)],
                    cnt_sh.at[pl.ds(sid * G_PER_SUB, G_PER_SUB)])
    plsc.subcore_barrier()
    _fill_rows(ones_v, CHP, H // LANES, 1.0)

    nper = NPCHUNKS // NW

    @pl.loop(0, nper + 1)
    def _(j):
        chunk = wid + NW * j

        @pl.when(chunk < NPCHUNKS)
        def _():
            base = chunk * CHP
            pltpu.sync_copy(h_hbm.at[pl.ds(base, CHP)], rows_v)
            pltpu.sync_copy(batch_hbm.at[pl.ds(base, CHP)], batch_v)
            pltpu.sync_copy(rows_v, sum_sh.at[batch_v], add=True)
            pltpu.sync_copy(ones_v, cnt_sh.at[batch_v], add=True)

    plsc.subcore_barrier()
    pltpu.sync_copy(sum_sh.at[pl.ds(sid * G_PER_SUB, G_PER_SUB)],
                    sum_hbm.at[cid, pl.ds(sid * G_PER_SUB, G_PER_SUB)])
    pltpu.sync_copy(cnt_sh.at[pl.ds(sid * G_PER_SUB, G_PER_SUB)],
                    cnt_hbm.at[cid, pl.ds(sid * G_PER_SUB, G_PER_SUB)])


_BN = 1000


def _dinv_body(d_ref, o_ref):
    deg = d_ref[0, :, 0:1] + d_ref[1, :, 0:1] + 1.0
    o_ref[...] = lax.rsqrt(deg)


def _tc_dinv(deg_parts):
    return pl.pallas_call(
        _dinv_body,
        grid=(N // _BN,),
        in_specs=[pl.BlockSpec((NC, _BN, DEGW), lambda i: (0, i, 0))],
        out_specs=pl.BlockSpec((_BN, 1), lambda i: (i, 0)),
        out_shape=jax.ShapeDtypeStruct((N, 1), jnp.float32),
    )(deg_parts)


def _projmm_body(x_ref, wi_ref, bi_ref, w_ref, s_ref, ho_ref, mo_ref):
    h = jnp.dot(x_ref[...], wi_ref[...],
                preferred_element_type=jnp.float32) + bi_ref[...]
    ho_ref[...] = h
    mo_ref[...] = jnp.dot(h, w_ref[...],
                          preferred_element_type=jnp.float32) * s_ref[...]


def _tc_projmm(x, w_in, b_in2, w0, dinv):
    return pl.pallas_call(
        _projmm_body,
        grid=(N // _BN,),
        in_specs=[
            pl.BlockSpec((_BN, H), lambda i: (i, 0)),
            pl.BlockSpec((H, H), lambda i: (0, 0)),
            pl.BlockSpec((1, H), lambda i: (0, 0)),
            pl.BlockSpec((H, H), lambda i: (0, 0)),
            pl.BlockSpec((_BN, 1), lambda i: (i, 0)),
        ],
        out_specs=[pl.BlockSpec((_BN, H), lambda i: (i, 0)),
                   pl.BlockSpec((_BN, H), lambda i: (i, 0))],
        out_shape=[jax.ShapeDtypeStruct((N, H), jnp.float32),
                   jax.ShapeDtypeStruct((N, H), jnp.float32)],
    )(x, w_in, b_in2, w0, dinv)


def _ln_relu_res(a_ref, m_ref, s_ref, cb_ref, g_ref, b_ref, hp_ref):
    t = (a_ref[0] + a_ref[1] + m_ref[...]) * s_ref[...] + cb_ref[...]
    mu = jnp.mean(t, axis=-1, keepdims=True)
    var = jnp.mean((t - mu) ** 2, axis=-1, keepdims=True)
    hn = (t - mu) * lax.rsqrt(var + 1e-5) * g_ref[...] + b_ref[...]
    return jnp.maximum(hn, 0.0) + hp_ref[...]


def _postmm_body(a_ref, m_ref, s_ref, cb_ref, g_ref, b_ref, hp_ref, w_ref,
                 ho_ref, mo_ref):
    h_new = _ln_relu_res(a_ref, m_ref, s_ref, cb_ref, g_ref, b_ref, hp_ref)
    ho_ref[...] = h_new
    mo_ref[...] = jnp.dot(h_new, w_ref[...],
                          preferred_element_type=jnp.float32) * s_ref[...]


def _tc_postmm(agg_parts, m2, dinv, conv_b_i, ln_g_i, ln_b_i, h_prev, w_next):
    return pl.pallas_call(
        _postmm_body,
        grid=(N // _BN,),
        in_specs=[
            pl.BlockSpec((NC, _BN, H), lambda i: (0, i, 0)),
            pl.BlockSpec((_BN, H), lambda i: (i, 0)),
            pl.BlockSpec((_BN, 1), lambda i: (i, 0)),
            pl.BlockSpec((1, H), lambda i: (0, 0)),
            pl.BlockSpec((1, H), lambda i: (0, 0)),
            pl.BlockSpec((1, H), lambda i: (0, 0)),
            pl.BlockSpec((_BN, H), lambda i: (i, 0)),
            pl.BlockSpec((H, H), lambda i: (0, 0)),
        ],
        out_specs=[pl.BlockSpec((_BN, H), lambda i: (i, 0)),
                   pl.BlockSpec((_BN, H), lambda i: (i, 0))],
        out_shape=[jax.ShapeDtypeStruct((N, H), jnp.float32),
                   jax.ShapeDtypeStruct((N, H), jnp.float32)],
    )(agg_parts, m2, dinv, conv_b_i, ln_g_i, ln_b_i, h_prev, w_next)


def _post_body(a_ref, m_ref, s_ref, cb_ref, g_ref, b_ref, hp_ref, o_ref):
    o_ref[...] = _ln_relu_res(a_ref, m_ref, s_ref, cb_ref, g_ref, b_ref,
                              hp_ref)


def _tc_post(agg_parts, m2, dinv, conv_b_i, ln_g_i, ln_b_i, h_prev):
    return pl.pallas_call(
        _post_body,
        grid=(N // _BN,),
        in_specs=[
            pl.BlockSpec((NC, _BN, H), lambda i: (0, i, 0)),
            pl.BlockSpec((_BN, H), lambda i: (i, 0)),
            pl.BlockSpec((_BN, 1), lambda i: (i, 0)),
            pl.BlockSpec((1, H), lambda i: (0, 0)),
            pl.BlockSpec((1, H), lambda i: (0, 0)),
            pl.BlockSpec((1, H), lambda i: (0, 0)),
            pl.BlockSpec((_BN, H), lambda i: (i, 0)),
        ],
        out_specs=pl.BlockSpec((_BN, H), lambda i: (i, 0)),
        out_shape=jax.ShapeDtypeStruct((N, H), jnp.float32),
    )(agg_parts, m2, dinv, conv_b_i, ln_g_i, ln_b_i, h_prev)


def _mlp_body(pp_ref, cp_ref, w1_ref, b1_ref, w2_ref, b2_ref, o_ref):
    sums = pp_ref[0] + pp_ref[1]
    cnts = cp_ref[0, :, 0:1] + cp_ref[1, :, 0:1]
    pooled = sums / jnp.maximum(cnts, 1.0)
    z = jnp.dot(pooled, w1_ref[...], preferred_element_type=jnp.float32)
    z = jnp.maximum(z + b1_ref[...], 0.0)
    o_ref[...] = jnp.dot(z, w2_ref[...],
                         preferred_element_type=jnp.float32) + b2_ref[...]


def _tc_mlp(pool_parts, cnt_parts, w1, b1, w2, b2):
    h2 = w1.shape[1]
    return pl.pallas_call(
        _mlp_body,
        grid=(1,),
        in_specs=[
            pl.BlockSpec((NC, G, H), lambda i: (0, 0, 0)),
            pl.BlockSpec((NC, G, H), lambda i: (0, 0, 0)),
            pl.BlockSpec((H, h2), lambda i: (0, 0)),
            pl.BlockSpec((1, h2), lambda i: (0, 0)),
            pl.BlockSpec((h2, 1), lambda i: (0, 0)),
            pl.BlockSpec((1, 1), lambda i: (0, 0)),
        ],
        out_specs=pl.BlockSpec((G, 1), lambda i: (0, 0)),
        out_shape=jax.ShapeDtypeStruct((G, 1), jnp.float32),
    )(pool_parts, cnt_parts, w1, b1, w2, b2)


def kernel(x, edge_index, batch, W_in, b_in, conv_W, conv_b, ln_g, ln_b,
           W1, b1, W2, b2):
    _sc_degree, _sc_edge_agg, _sc_pool = _sc_kernels()

    k = jnp.arange(E_PAD - E, dtype=jnp.int32)
    pad = jnp.stack([k % 1024, N + (k % NSPARE)])
    e3 = (jnp.concatenate([edge_index, pad], axis=1)
          .reshape(2, EPC, NW, CH).swapaxes(1, 2)
          .reshape(2, NCHUNKS_P, CH))

    deg_parts = _sc_degree(e3)
    dinv = _tc_dinv(deg_parts)

    h, m2 = _tc_projmm(x, W_in, b_in.reshape(1, H), conv_W[0], dinv)
    for i in range(L):
        agg_parts = _sc_edge_agg(m2, e3)
        args = (agg_parts, m2, dinv, conv_b[i].reshape(1, H),
                ln_g[i].reshape(1, H), ln_b[i].reshape(1, H), h)
        if i < L - 1:
            h, m2 = _tc_postmm(*args, conv_W[i + 1])
        else:
            h = _tc_post(*args)

    pool_parts, cnt_parts = _sc_pool(h, batch)
    out = _tc_mlp(pool_parts, cnt_parts, W1, b1.reshape(1, -1), W2,
                  b2.reshape(1, 1))
    return out
